# Initial kernel scaffold; baseline (speedup 1.0000x reference)
import jax
import jax.numpy as jnp
from jax.experimental import pallas as pl


def kernel(x, edge_index, batch, W_in, b_in, g_in, bt_in, We, be, ge, bte, Wc, gn, btn, W1, b1, g1, bt1, W2, b2):
    raise NotImplementedError("write your pallas kernel here")


# speedup vs baseline: 10.5554x; 10.5554x over previous
import functools

import jax
import jax.numpy as jnp
from jax import lax
from jax.experimental import pallas as pl
from jax.experimental.pallas import tpu as pltpu
from jax.experimental.pallas import tpu_sc as plsc

N = 10000
E = 320000
D = 128
G = 64
EPS = 1e-5

NC = 2
NS = 16
NW = NC * NS

CHUNK = 128
NCHUNK = 80
NB = 8
NBLK = NCHUNK // NB
T_EDGES = CHUNK * NCHUNK
EPAD = T_EDGES * NW
NSCRAP = 112
NPAD = N + NSCRAP
ROWS_PER_TILE = NPAD // NS


@functools.partial(
    pl.kernel,
    out_type=jax.ShapeDtypeStruct((NC, NPAD, D), jnp.float32),
    mesh=plsc.VectorSubcoreMesh(core_axis_name="c", subcore_axis_name="s",
                                num_cores=NC, num_subcores=NS),
    scratch_types=[
        pltpu.VMEM((2, NB, CHUNK), jnp.int32),
        pltpu.VMEM((2, NB, CHUNK), jnp.int32),
        pltpu.VMEM((2, CHUNK, D), jnp.float32),
        pltpu.VMEM_SHARED((NPAD, D), jnp.float32),
        pltpu.SemaphoreType.DMA,
        pltpu.SemaphoreType.DMA,
        pltpu.SemaphoreType.DMA,
        pltpu.SemaphoreType.DMA,
    ],
)
def _sc_agg(m_hbm, srcp_hbm, dstp_hbm, zeros_hbm, out_hbm,
            src_v, dst_v, rows_v, agg_sh, sem_is, sem_id, sem_r0, sem_r1):
    c = lax.axis_index("c")
    s = lax.axis_index("s")
    wid = c * NS + s
    base = wid * NCHUNK
    sem_r = (sem_r0, sem_r1)

    pltpu.sync_copy(srcp_hbm.at[pl.ds(base, NB)], src_v.at[0])
    pltpu.sync_copy(dstp_hbm.at[pl.ds(base, NB)], dst_v.at[0])

    zr = s * ROWS_PER_TILE
    pltpu.sync_copy(zeros_hbm.at[pl.ds(zr, ROWS_PER_TILE)],
                    agg_sh.at[pl.ds(zr, ROWS_PER_TILE)])
    plsc.subcore_barrier()

    pltpu.async_copy(m_hbm.at[src_v.at[0, 0]], rows_v.at[0], sem_r0)

    def blk(b, _):
        bs = b % 2
        ns = (b + 1) % 2
        nxt = pl.ds(base + (b + 1) * NB, NB)

        @pl.when(b + 1 < NBLK)
        def _():
            pltpu.async_copy(srcp_hbm.at[nxt], src_v.at[ns], sem_is)
            pltpu.async_copy(dstp_hbm.at[nxt], dst_v.at[ns], sem_id)

        for k in range(NB):
            rs = k % 2
            rn = (k + 1) % 2
            pltpu.make_async_copy(m_hbm.at[src_v.at[bs, k]], rows_v.at[rs],
                                  sem_r[rs]).wait()
            if k < NB - 1:
                pltpu.async_copy(m_hbm.at[src_v.at[bs, k + 1]],
                                 rows_v.at[rn], sem_r[rn])
            else:
                @pl.when(b + 1 < NBLK)
                def _():
                    pltpu.make_async_copy(srcp_hbm.at[nxt], src_v.at[ns],
                                          sem_is).wait()
                    pltpu.make_async_copy(dstp_hbm.at[nxt], dst_v.at[ns],
                                          sem_id).wait()
                    pltpu.async_copy(m_hbm.at[src_v.at[ns, 0]],
                                     rows_v.at[rn], sem_r[rn])
            pltpu.sync_copy(rows_v.at[rs], agg_sh.at[dst_v.at[bs, k]],
                            add=True)
        return 0

    lax.fori_loop(0, NBLK, blk, 0)

    plsc.subcore_barrier()
    pltpu.sync_copy(agg_sh.at[pl.ds(zr, ROWS_PER_TILE)],
                    out_hbm.at[c].at[pl.ds(zr, ROWS_PER_TILE)])


def _mm_t(a, w):
    return lax.dot_general(a, w, (((1,), (1,)), ((), ())),
                           preferred_element_type=jnp.float32)


def _col_mean(y, rows):
    ones = jnp.ones((8, rows), jnp.float32)
    s = lax.dot_general(ones, y, (((1,), (0,)), ((), ())),
                        precision=lax.Precision.HIGHEST,
                        preferred_element_type=jnp.float32)
    return s[0:1, :] / rows


def _bn_relu(y, g, bt):
    rows = y.shape[0]
    mu = _col_mean(y, rows)
    d = y - mu
    va = _col_mean(d * d, rows)
    return jnp.maximum(d / jnp.sqrt(va + EPS) * g + bt, 0.0)


def _in_body(x_ref, w_ref, b_ref, g_ref, bt_ref, c_ref, h_ref, m_ref):
    y = _mm_t(x_ref[:], w_ref[:]) + b_ref[:]
    h = _bn_relu(y, g_ref[:], bt_ref[:])
    h_ref[:] = h
    m_ref[:] = jnp.maximum(h + jnp.maximum(c_ref[:], 0.0), 0.0)


_tc_in = pl.pallas_call(
    _in_body,
    out_shape=[jax.ShapeDtypeStruct((N, D), jnp.float32),
               jax.ShapeDtypeStruct((N, D), jnp.float32)],
)


def _layer_body(h_ref, p_ref, w_ref, g_ref, bt_ref, c_ref, ho_ref, m_ref):
    hp = h_ref[:] + p_ref[0, :N, :] + p_ref[1, :N, :]
    t = _bn_relu(_mm_t(hp, w_ref[:]), g_ref[:], bt_ref[:])
    hn = t + h_ref[:]
    ho_ref[:] = hn
    m_ref[:] = jnp.maximum(hn + jnp.maximum(c_ref[:], 0.0), 0.0)


_tc_layer = pl.pallas_call(
    _layer_body,
    out_shape=[jax.ShapeDtypeStruct((N, D), jnp.float32),
               jax.ShapeDtypeStruct((N, D), jnp.float32)],
)


def _out_body(h_ref, p_ref, w_ref, g_ref, bt_ref, b_ref,
              w1_ref, b1_ref, g1_ref, bt1_ref, w2_ref, b2_ref, o_ref):
    hp = h_ref[:] + p_ref[0, :N, :] + p_ref[1, :N, :]
    t = _bn_relu(_mm_t(hp, w_ref[:]), g_ref[:], bt_ref[:])
    hn = t + h_ref[:]
    mask = (b_ref[:] == lax.broadcasted_iota(jnp.int32, (N, G), 1))
    pooled = lax.dot_general(mask.astype(jnp.float32), hn,
                             (((0,), (0,)), ((), ())),
                             precision=lax.Precision.HIGHEST,
                             preferred_element_type=jnp.float32)
    o = _bn_relu(_mm_t(pooled, w1_ref[:]) + b1_ref[:], g1_ref[:], bt1_ref[:])
    o_ref[:] = _mm_t(o, w2_ref[:]) + b2_ref[:]


_tc_out = pl.pallas_call(
    _out_body,
    out_shape=jax.ShapeDtypeStruct((G, D), jnp.float32),
)


def kernel(x, edge_index, batch, W_in, b_in, g_in, bt_in, We, be, ge, bte,
           Wc, gn, btn, W1, b1, g1, bt1, W2, b2):
    src = edge_index[0]
    dst = edge_index[1]
    npad = EPAD - E
    pad_i = jnp.arange(npad, dtype=jnp.int32)
    srcp = jnp.concatenate([src, (pad_i * 89) % N]).reshape(EPAD // CHUNK, CHUNK)
    dstp = jnp.concatenate([dst, N + pad_i % NSCRAP]).reshape(EPAD // CHUNK, CHUNK)
    zeros = jnp.zeros((NPAD, D), jnp.float32)
    batch2 = batch.reshape(N, 1)

    r = lambda v: v.reshape(1, D)

    h, m = _tc_in(x, W_in, r(b_in), r(g_in), r(bt_in), r(bte[0]))
    for i in range(2):
        parts = _sc_agg(m, srcp, dstp, zeros)
        h, m = _tc_layer(h, parts, Wc[i], r(gn[i]), r(btn[i]), r(bte[i + 1]))
    parts = _sc_agg(m, srcp, dstp, zeros)
    return _tc_out(h, parts, Wc[2], r(gn[2]), r(btn[2]), batch2,
                   W1, r(b1), r(g1), r(bt1), W2, r(b2))
